# baseline (speedup 1.0000x reference)
import jax
import jax.numpy as jnp
from jax import lax
from jax.experimental import pallas as pl
from jax.experimental.pallas import tpu as pltpu
from jax.experimental.pallas import tpu_sc as plsc

K = 5
B, S, D = 4, 8192, 1024
NC, NS, L = 2, 16, 16
NW = NC * NS
COLS = 128
NCG = COLS // L
G = 16
BLK = 256
NGRP_B = BLK // G
NBLK = S // BLK
NEG_INF = float("-inf")


def _insert(ms, v):
    out = []
    carry = v
    for m in ms:
        out.append(jnp.maximum(m, carry))
        carry = jnp.minimum(m, carry)
    return out


def _insert_with_idx(vals, idxs, v, vi):
    nv, ni = [], []
    cv, ci = v, vi
    for m, mi in zip(vals, idxs):
        take = cv > m
        nv.append(jnp.maximum(m, cv))
        ni.append(jnp.where(take, ci, mi))
        cv = jnp.minimum(m, cv)
        ci = jnp.where(take, mi, ci)
    return nv, ni


def _kmax_body(x_hbm, out_hbm, buf, cval, cgid, gbuf, ostage, sem, gsem):
    cid = lax.axis_index("c")
    sid = lax.axis_index("s")
    wid = sid * NC + cid
    lane = jnp.arange(L, dtype=jnp.int32)
    b = wid // (D // COLS)
    c0 = (wid % (D // COLS)) * COLS

    def src_slice(q):
        return x_hbm.at[pl.ds(b * S + q * BLK, BLK), pl.ds(c0, COLS)]

    def init_cand(i, _):
        cval[i, :] = jnp.full((L,), NEG_INF, jnp.float32)
        cgid[i, :] = jnp.zeros((L,), jnp.int32)
        return 0

    lax.fori_loop(0, NCG * K, init_cand, 0)

    pltpu.async_copy(src_slice(0), buf.at[0], sem.at[0])

    def step(q, _):
        par = q % 2
        pltpu.make_async_copy(src_slice(q), buf.at[par], sem.at[par]).wait()

        @pl.when(q + 1 < NBLK)
        def _():
            pltpu.async_copy(
                src_slice(q + 1), buf.at[1 - par], sem.at[1 - par]
            )

        for cg in range(0):
            carry0 = tuple(cval[cg * K + j, :] for j in range(K)) + tuple(
                cgid[cg * K + j, :] for j in range(K)
            )

            def grp(g, carry, cg=cg):
                vals, idxs = list(carry[:K]), list(carry[K:])
                r0 = g * G
                vs = [buf[par, r0 + r, pl.ds(cg * L, L)] for r in range(G)]
                while len(vs) > 1:
                    vs = [
                        jnp.maximum(vs[2 * i], vs[2 * i + 1])
                        for i in range(len(vs) // 2)
                    ]
                gid = jnp.zeros((L,), jnp.int32) + (q * NGRP_B + g)
                vals, idxs = _insert_with_idx(vals, idxs, vs[0], gid)
                return tuple(vals) + tuple(idxs)

            res = lax.fori_loop(0, NGRP_B, grp, carry0, unroll=2)
            for j in range(K):
                cval[cg * K + j, :] = res[j]
                cgid[cg * K + j, :] = res[K + j]
        return 0

    lax.fori_loop(0, NBLK, step, 0)

    def cand_copy(cg, j, slot, ell):
        g_s = cgid[cg * K + j, :][ell]
        return pltpu.make_async_copy(
            x_hbm.at[pl.ds(b * S + g_s * G, G), pl.ds(c0 + cg * L, L)],
            gbuf.at[slot, ell],
            gsem,
        )

    def issue(cg, j, slot):
        for ell in range(L):
            cand_copy(cg, j, slot, ell).start()

    for cg in range(NCG):
        issue(cg, 0, 0)
        state0 = tuple(jnp.full((L,), NEG_INF, jnp.float32) for _ in range(K))

        def cand_j(j, state, cg=cg):
            slot = j % 2
            for ell in range(L):
                cand_copy(cg, j, slot, ell).wait()

            @pl.when(j + 1 < K)
            def _():
                issue(cg, j + 1, 1 - slot)

            def row_r(r, state):
                v = plsc.load_gather(
                    gbuf, [jnp.zeros((L,), jnp.int32) + slot, lane,
                           lane * 0 + r, lane]
                )
                return tuple(_insert(list(state), v))

            return lax.fori_loop(0, G, row_r, state)

        state = lax.fori_loop(0, K, cand_j, state0)
        for j in range(K):
            plsc.store_scatter(
                ostage, [(lane + cg * L) * K + j], state[j]
            )

    pltpu.sync_copy(ostage, out_hbm.at[pl.ds(b * D * K + c0 * K, COLS * K)])


@jax.jit
def kernel(inputs):
    mesh = plsc.VectorSubcoreMesh(
        core_axis_name="c", subcore_axis_name="s", num_cores=NC,
        num_subcores=NS,
    )
    kfn = pl.kernel(
        _kmax_body,
        out_type=jax.ShapeDtypeStruct((B * D * K,), jnp.float32),
        mesh=mesh,
        scratch_types=[
            pltpu.VMEM((2, BLK, COLS), jnp.float32),
            pltpu.VMEM((NCG * K, L), jnp.float32),
            pltpu.VMEM((NCG * K, L), jnp.int32),
            pltpu.VMEM((2, L, G, L), jnp.float32),
            pltpu.VMEM((COLS * K,), jnp.float32),
            pltpu.SemaphoreType.DMA((2,)),
            pltpu.SemaphoreType.DMA,
        ],
        compiler_params=pltpu.CompilerParams(
            use_tc_tiling_on_sc=False, needs_layout_passes=False
        ),
    )
    out = kfn(inputs.reshape(B * S, D))
    return out.reshape(B, D * K)


# speedup vs baseline: 1.5604x; 1.5604x over previous
import jax
import jax.numpy as jnp
from jax import lax
from jax.experimental import pallas as pl
from jax.experimental.pallas import tpu as pltpu
from jax.experimental.pallas import tpu_sc as plsc

K = 5
B, S, D = 4, 8192, 1024
NC, NS, L = 2, 16, 16
NW = NC * NS
COLS = 128
NCG = COLS // L
G = 16
BLK = 256
NGRP_B = BLK // G
NBLK = S // BLK
NEG_INF = float("-inf")


def _insert(ms, v):
    out = []
    carry = v
    for m in ms:
        out.append(jnp.maximum(m, carry))
        carry = jnp.minimum(m, carry)
    return out


def _insert_with_idx(vals, idxs, v, vi):
    nv, ni = [], []
    cv, ci = v, vi
    for m, mi in zip(vals, idxs):
        take = cv > m
        nv.append(jnp.maximum(m, cv))
        ni.append(jnp.where(take, ci, mi))
        cv = jnp.minimum(m, cv)
        ci = jnp.where(take, mi, ci)
    return nv, ni


def _kmax_body(x_hbm, out_hbm, buf, sval, ostage, sem):
    cid = lax.axis_index("c")
    sid = lax.axis_index("s")
    wid = sid * NC + cid
    lane = jnp.arange(L, dtype=jnp.int32)
    b = wid // (D // COLS)
    c0 = (wid % (D // COLS)) * COLS

    def src_slice(q):
        return x_hbm.at[pl.ds(b * S + q * BLK, BLK), pl.ds(c0, COLS)]

    def init_state(i, _):
        sval[i, :] = jnp.full((L,), NEG_INF, jnp.float32)
        return 0

    lax.fori_loop(0, NCG * K, init_state, 0)

    pltpu.async_copy(src_slice(0), buf.at[0], sem.at[0])

    def step(q, _):
        par = q % 2
        pltpu.make_async_copy(src_slice(q), buf.at[par], sem.at[par]).wait()

        @pl.when(q + 1 < NBLK)
        def _():
            pltpu.async_copy(
                src_slice(q + 1), buf.at[1 - par], sem.at[1 - par]
            )

        pv = jnp.zeros((L,), jnp.int32) + par
        for cg in range(NCG):
            cinit = tuple(
                jnp.full((L,), NEG_INF, jnp.float32) for _ in range(K)
            ) + tuple(jnp.zeros((L,), jnp.int32) for _ in range(K))

            def grp(g, carry, cg=cg):
                vals, idxs = list(carry[:K]), list(carry[K:])
                r0 = g * G
                vs = [buf[par, r0 + r, pl.ds(cg * L, L)] for r in range(G)]
                while len(vs) > 1:
                    vs = [
                        jnp.maximum(vs[2 * i], vs[2 * i + 1])
                        for i in range(len(vs) // 2)
                    ]
                gid = jnp.zeros((L,), jnp.int32) + g
                vals, idxs = _insert_with_idx(vals, idxs, vs[0], gid)
                return tuple(vals) + tuple(idxs)

            cand = lax.fori_loop(0, NGRP_B, grp, cinit, unroll=2)

            state = tuple(sval[cg * K + j, :] for j in range(K))
            col = cg * L + lane

            def cand_j(j, st, cg=cg):
                gid = jnp.where(
                    j == 0, cand[K],
                    jnp.where(
                        j == 1, cand[K + 1],
                        jnp.where(
                            j == 2, cand[K + 2],
                            jnp.where(j == 3, cand[K + 3], cand[K + 4]),
                        ),
                    ),
                )
                base = gid * G

                def row_r(r, st):
                    v = plsc.load_gather(buf, [pv, base + r, col])
                    return tuple(_insert(list(st), v))

                return lax.fori_loop(0, G, row_r, st)

            state = lax.fori_loop(0, K, cand_j, state)
            for j in range(K):
                sval[cg * K + j, :] = state[j]
        return 0

    lax.fori_loop(0, NBLK, step, 0)

    for cg in range(NCG):
        for j in range(K):
            plsc.store_scatter(
                ostage, [(lane + cg * L) * K + j], sval[cg * K + j, :]
            )
    pltpu.sync_copy(ostage, out_hbm.at[pl.ds(b * D * K + c0 * K, COLS * K)])


@jax.jit
def kernel(inputs):
    mesh = plsc.VectorSubcoreMesh(
        core_axis_name="c", subcore_axis_name="s", num_cores=NC,
        num_subcores=NS,
    )
    kfn = pl.kernel(
        _kmax_body,
        out_type=jax.ShapeDtypeStruct((B * D * K,), jnp.float32),
        mesh=mesh,
        scratch_types=[
            pltpu.VMEM((2, BLK, COLS), jnp.float32),
            pltpu.VMEM((NCG * K, L), jnp.float32),
            pltpu.VMEM((COLS * K,), jnp.float32),
            pltpu.SemaphoreType.DMA((2,)),
        ],
        compiler_params=pltpu.CompilerParams(
            use_tc_tiling_on_sc=True, needs_layout_passes=False
        ),
    )
    out = kfn(inputs.reshape(B * S, D))
    return out.reshape(B, D * K)


# speedup vs baseline: 1.5605x; 1.0001x over previous
import jax
import jax.numpy as jnp
from jax import lax
from jax.experimental import pallas as pl
from jax.experimental.pallas import tpu as pltpu
from jax.experimental.pallas import tpu_sc as plsc

K = 5
B, S, D = 4, 8192, 1024
NC, NS, L = 2, 16, 16
NW = NC * NS
COLS = 128
NCG = COLS // L
G = 16
BLK = 256
NGRP_B = BLK // G
NBLK = S // BLK
NEG_INF = float("-inf")


def _insert(ms, v):
    out = []
    carry = v
    for m in ms:
        out.append(jnp.maximum(m, carry))
        carry = jnp.minimum(m, carry)
    return out


def _insert_with_idx(vals, idxs, v, vi):
    nv, ni = [], []
    cv, ci = v, vi
    for m, mi in zip(vals, idxs):
        take = cv > m
        nv.append(jnp.maximum(m, cv))
        ni.append(jnp.where(take, ci, mi))
        cv = jnp.minimum(m, cv)
        ci = jnp.where(take, mi, ci)
    return nv, ni


def _kmax_body(x_hbm, out_hbm, buf, sval, ostage, sem):
    cid = lax.axis_index("c")
    sid = lax.axis_index("s")
    wid = sid * NC + cid
    lane = jnp.arange(L, dtype=jnp.int32)
    b = wid // (D // COLS)
    c0 = (wid % (D // COLS)) * COLS

    H = BLK // 2

    def src_half(q, h):
        return x_hbm.at[
            pl.ds(b * S + q * BLK + h * H, H), pl.ds(c0, COLS)
        ]

    def start_block(q, par):
        for h in range(2):
            pltpu.async_copy(
                src_half(q, h), buf.at[par, pl.ds(h * H, H)], sem.at[2 * par + h]
            )

    def wait_block(q, par):
        for h in range(2):
            pltpu.make_async_copy(
                src_half(q, h), buf.at[par, pl.ds(h * H, H)],
                sem.at[2 * par + h],
            ).wait()

    def init_state(i, _):
        sval[i, :] = jnp.full((L,), NEG_INF, jnp.float32)
        return 0

    lax.fori_loop(0, NCG * K, init_state, 0)

    start_block(0, 0)

    def step(q, _):
        par = q % 2
        wait_block(q, par)

        @pl.when(q + 1 < NBLK)
        def _():
            start_block(q + 1, 1 - par)

        pv = jnp.zeros((L,), jnp.int32) + par
        for cg in range(NCG):
            cinit = tuple(
                jnp.full((L,), NEG_INF, jnp.float32) for _ in range(K)
            ) + tuple(jnp.zeros((L,), jnp.int32) for _ in range(K))

            def grp(g, carry, cg=cg):
                vals, idxs = list(carry[:K]), list(carry[K:])
                r0 = g * G
                vs = [buf[par, r0 + r, pl.ds(cg * L, L)] for r in range(G)]
                while len(vs) > 1:
                    vs = [
                        jnp.maximum(vs[2 * i], vs[2 * i + 1])
                        for i in range(len(vs) // 2)
                    ]
                gid = jnp.zeros((L,), jnp.int32) + g
                vals, idxs = _insert_with_idx(vals, idxs, vs[0], gid)
                return tuple(vals) + tuple(idxs)

            cand = lax.fori_loop(0, NGRP_B, grp, cinit, unroll=2)

            state = tuple(sval[cg * K + j, :] for j in range(K))
            col = cg * L + lane

            def cand_j(j, st, cg=cg):
                gid = jnp.where(
                    j == 0, cand[K],
                    jnp.where(
                        j == 1, cand[K + 1],
                        jnp.where(
                            j == 2, cand[K + 2],
                            jnp.where(j == 3, cand[K + 3], cand[K + 4]),
                        ),
                    ),
                )
                base = gid * G

                def row_r(r, st):
                    v = plsc.load_gather(buf, [pv, base + r, col])
                    return tuple(_insert(list(st), v))

                return lax.fori_loop(0, G, row_r, st)

            state = lax.fori_loop(0, K, cand_j, state)
            for j in range(K):
                sval[cg * K + j, :] = state[j]
        return 0

    lax.fori_loop(0, NBLK, step, 0)

    for cg in range(NCG):
        for j in range(K):
            plsc.store_scatter(
                ostage, [(lane + cg * L) * K + j], sval[cg * K + j, :]
            )
    pltpu.sync_copy(ostage, out_hbm.at[pl.ds(b * D * K + c0 * K, COLS * K)])


@jax.jit
def kernel(inputs):
    mesh = plsc.VectorSubcoreMesh(
        core_axis_name="c", subcore_axis_name="s", num_cores=NC,
        num_subcores=NS,
    )
    kfn = pl.kernel(
        _kmax_body,
        out_type=jax.ShapeDtypeStruct((B * D * K,), jnp.float32),
        mesh=mesh,
        scratch_types=[
            pltpu.VMEM((2, BLK, COLS), jnp.float32),
            pltpu.VMEM((NCG * K, L), jnp.float32),
            pltpu.VMEM((COLS * K,), jnp.float32),
            pltpu.SemaphoreType.DMA((4,)),
        ],
        compiler_params=pltpu.CompilerParams(
            use_tc_tiling_on_sc=True, needs_layout_passes=False
        ),
    )
    out = kfn(inputs.reshape(B * S, D))
    return out.reshape(B, D * K)
